# Initial kernel scaffold; baseline (speedup 1.0000x reference)
import jax
import jax.numpy as jnp
from jax.experimental import pallas as pl


def kernel(x, gate_w, w_gate, w_up, w_down):
    raise NotImplementedError("write your pallas kernel here")


# speedup vs baseline: 1.6923x; 1.6923x over previous
import functools

import jax
import jax.numpy as jnp
from jax.experimental import pallas as pl
from jax.experimental.pallas import tpu as pltpu

_B, _S, _D, _FF, _E, _TOP_K = 1, 2048, 768, 2048, 8, 2
_FFB = 512
_NFF = _FF // _FFB


def _router_body(x_ref, gw_ref, comb_ref, aux_ref):
    xf = x_ref[...]
    logits = jax.lax.dot_general(
        xf, gw_ref[...], (((1,), (1,)), ((), ())),
        preferred_element_type=jnp.float32)
    n = logits.shape[0]
    iota = jax.lax.broadcasted_iota(jnp.int32, logits.shape, 1)
    m1 = jnp.max(logits, axis=1, keepdims=True)
    i1 = jnp.min(jnp.where(logits == m1, iota, _E), axis=1, keepdims=True)
    sel1 = iota == i1
    masked = jnp.where(sel1, -jnp.inf, logits)
    m2 = jnp.max(masked, axis=1, keepdims=True)
    i2 = jnp.min(jnp.where(masked == m2, iota, _E), axis=1, keepdims=True)
    sel2 = iota == i2
    w1 = 1.0 / (1.0 + jnp.exp(m2 - m1))
    w2 = 1.0 - w1
    comb_ref[...] = jnp.where(sel1, w1, 0.0) + jnp.where(sel2, w2, 0.0)
    ez = jnp.exp(logits - m1)
    probs = ez / jnp.sum(ez, axis=1, keepdims=True)
    p_mean = jnp.sum(probs, axis=0, keepdims=True) / n
    counts = jnp.sum(sel1.astype(jnp.float32) + sel2.astype(jnp.float32),
                     axis=0, keepdims=True)
    aux_ref[0, 0] = _E * jnp.sum((counts / n) * p_mean)


def _ffn_body(comb_ref, x_ref, wg_ref, wu_ref, wd_ref, out_ref):
    e = pl.program_id(0)
    fb = pl.program_id(1)
    x = x_ref[...]
    g = jax.lax.dot_general(x, wg_ref[0], (((1,), (1,)), ((), ())),
                            preferred_element_type=jnp.float32)
    u = jax.lax.dot_general(x, wu_ref[0], (((1,), (1,)), ((), ())),
                            preferred_element_type=jnp.float32)
    h = (g * jax.nn.sigmoid(g)) * u
    y = jax.lax.dot_general(h, wd_ref[0], (((1,), (1,)), ((), ())),
                            preferred_element_type=jnp.float32)
    comb = comb_ref[...]
    eio = jax.lax.broadcasted_iota(jnp.int32, comb.shape, 1)
    c = jnp.sum(jnp.where(eio == e, comb, 0.0), axis=1, keepdims=True)
    contrib = c * y
    first = jnp.logical_and(e == 0, fb == 0)

    @pl.when(first)
    def _():
        out_ref[...] = contrib

    @pl.when(jnp.logical_not(first))
    def _():
        out_ref[...] += contrib


@jax.jit
def kernel(x, gate_w, w_gate, w_up, w_down):
    n = _B * _S
    flat = x.reshape(n, _D)

    comb, aux = pl.pallas_call(
        _router_body,
        out_shape=(
            jax.ShapeDtypeStruct((n, _E), jnp.float32),
            jax.ShapeDtypeStruct((1, 1), jnp.float32),
        ),
        in_specs=[
            pl.BlockSpec(memory_space=pltpu.VMEM),
            pl.BlockSpec(memory_space=pltpu.VMEM),
        ],
        out_specs=(
            pl.BlockSpec(memory_space=pltpu.VMEM),
            pl.BlockSpec(memory_space=pltpu.SMEM),
        ),
    )(flat, gate_w)

    out = pl.pallas_call(
        _ffn_body,
        grid=(_E, _NFF),
        in_specs=[
            pl.BlockSpec((n, _E), lambda e, f: (0, 0)),
            pl.BlockSpec((n, _D), lambda e, f: (0, 0)),
            pl.BlockSpec((1, _FFB, _D), lambda e, f: (e, f, 0)),
            pl.BlockSpec((1, _FFB, _D), lambda e, f: (e, f, 0)),
            pl.BlockSpec((1, _D, _FFB), lambda e, f: (e, 0, f)),
        ],
        out_specs=pl.BlockSpec((n, _D), lambda e, f: (0, 0)),
        out_shape=jax.ShapeDtypeStruct((n, _D), jnp.float32),
    )(comb, flat, w_gate, w_up, w_down)

    return out.reshape(_B, _S, _D), aux.reshape(())
